# Initial kernel scaffold; baseline (speedup 1.0000x reference)
import jax
import jax.numpy as jnp
from jax.experimental import pallas as pl


def kernel(indices, points, sampleLocs):
    raise NotImplementedError("write your pallas kernel here")


# speedup vs baseline: 1.6583x; 1.6583x over previous
import functools

import jax
import jax.numpy as jnp
from jax.experimental import pallas as pl
from jax.experimental.pallas import tpu as pltpu
from jax.experimental.pallas import tpu_sc as plsc

B = 4096
K = 50
M = 100000
D = 128
L = 16
ND = D // L
WIN_B = 2
WIN = WIN_B * K
GRID = B // WIN_B


def _make_sc_kernel():
    mesh = plsc.VectorSubcoreMesh(
        core_axis_name="core", subcore_axis_name="subcore"
    )

    @functools.partial(
        pl.kernel,
        out_type=jax.ShapeDtypeStruct((B * K, D), jnp.float32),
        mesh=mesh,
        compiler_params=pltpu.CompilerParams(use_tc_tiling_on_sc=False),
    )
    def run(table_hbm, idx_hbm, pts_hbm, out_hbm):
        def body(i_vmem, p_vmem, o_vmem):
            pltpu.sync_copy(table_hbm.at[i_vmem.at[0, 0]], o_vmem)
            for b in range(WIN_B):
                pvecs = [p_vmem[b, pl.ds(d * L, L)] for d in range(ND)]

                @pl.loop(0, K)
                def _(k, b=b, pvecs=pvecs):
                    r = b * K + k
                    for d in range(ND):
                        sl = pl.ds(d * L, L)
                        o_vmem[r, sl] = o_vmem[r, sl] - pvecs[d]

        pltpu.emit_pipeline(
            body,
            grid=(GRID,),
            in_specs=[
                pl.BlockSpec((1, 1, WIN), lambda i: (i, 0, 0)),
                pl.BlockSpec((WIN_B, D), lambda i: (i, 0)),
            ],
            out_specs=[pl.BlockSpec((WIN, D), lambda i: (i, 0))],
            core_axis_name=("core", "subcore"),
            dimension_semantics=(pltpu.PARALLEL,),
        )(idx_hbm, pts_hbm, out_hbm)

    return run


_sc_kernel = _make_sc_kernel()


def kernel(indices, points, sampleLocs):
    idx_flat = indices.astype(jnp.int32).reshape(GRID, 1, WIN)
    out = _sc_kernel(sampleLocs, idx_flat, points)
    return out.reshape(B, K, D)


# speedup vs baseline: 1.7365x; 1.0472x over previous
import functools

import jax
import jax.numpy as jnp
from jax.experimental import pallas as pl
from jax.experimental.pallas import tpu as pltpu
from jax.experimental.pallas import tpu_sc as plsc

B = 4096
K = 50
M = 100000
D = 128
L = 16
ND = D // L
WIN_B = 2
WIN = WIN_B * K
GRID = B // WIN_B


def _make_sc_kernel():
    mesh = plsc.VectorSubcoreMesh(
        core_axis_name="core", subcore_axis_name="subcore"
    )

    @functools.partial(
        pl.kernel,
        out_type=jax.ShapeDtypeStruct((B, K, D), jnp.float32),
        mesh=mesh,
        scratch_types=[pltpu.VMEM((WIN, D), jnp.float32)],
        compiler_params=pltpu.CompilerParams(use_tc_tiling_on_sc=False),
    )
    def run(table_hbm, idx_hbm, pts_hbm, out_hbm, rows_v):
        def body(i_vmem, p_vmem, o_vmem):
            pltpu.sync_copy(table_hbm.at[i_vmem.at[0, 0]], rows_v)
            for b in range(WIN_B):
                pvecs = [p_vmem[b, pl.ds(d * L, L)] for d in range(ND)]

                @pl.loop(0, K)
                def _(k, b=b, pvecs=pvecs):
                    r = b * K + k
                    for d in range(ND):
                        sl = pl.ds(d * L, L)
                        o_vmem[b, k, sl] = rows_v[r, sl] - pvecs[d]

        pltpu.emit_pipeline(
            body,
            grid=(GRID,),
            in_specs=[
                pl.BlockSpec((1, 1, WIN), lambda i: (i, 0, 0)),
                pl.BlockSpec((WIN_B, D), lambda i: (i, 0)),
            ],
            out_specs=[pl.BlockSpec((WIN_B, K, D), lambda i: (i, 0, 0))],
            core_axis_name=("core", "subcore"),
            dimension_semantics=(pltpu.PARALLEL,),
        )(idx_hbm, pts_hbm, out_hbm)

    return run


_sc_kernel = _make_sc_kernel()


def kernel(indices, points, sampleLocs):
    idx_flat = indices.astype(jnp.int32).reshape(GRID, 1, WIN)
    return _sc_kernel(sampleLocs, idx_flat, points)


# speedup vs baseline: 3.5798x; 2.0615x over previous
import functools

import jax
import jax.numpy as jnp
from jax import lax
from jax.experimental import pallas as pl
from jax.experimental.pallas import tpu as pltpu
from jax.experimental.pallas import tpu_sc as plsc

B = 4096
K = 50
M = 100000
D = 128
L = 16
ND = D // L
NW = 32
WIN_B = 2
WIN = WIN_B * K
BPW = B // NW
CH = BPW // WIN_B


def _make_sc_kernel():
    mesh = plsc.VectorSubcoreMesh(
        core_axis_name="core", subcore_axis_name="subcore"
    )

    @functools.partial(
        pl.kernel,
        out_type=jax.ShapeDtypeStruct((K, B, D), jnp.float32),
        mesh=mesh,
        scratch_types=[
            pltpu.VMEM((CH, 1, WIN), jnp.int32),
            pltpu.VMEM((BPW, D), jnp.float32),
            pltpu.VMEM((WIN, D), jnp.float32),
            pltpu.VMEM((WIN, D), jnp.float32),
            pltpu.VMEM((K, WIN_B, D), jnp.float32),
            pltpu.VMEM((K, WIN_B, D), jnp.float32),
            pltpu.SemaphoreType.DMA,
            pltpu.SemaphoreType.DMA,
            pltpu.SemaphoreType.DMA,
            pltpu.SemaphoreType.DMA,
        ],
        compiler_params=pltpu.CompilerParams(use_tc_tiling_on_sc=False),
    )
    def run(table_hbm, idx_hbm, pts_hbm, out_hbm,
            idx_v, pts_v, r0, r1, o0, o1, gs0, gs1, os0, os1):
        rows = (r0, r1)
        obuf = (o0, o1)
        gsem = (gs0, gs1)
        osem = (os0, os1)

        wid = lax.axis_index("core") * 16 + lax.axis_index("subcore")

        pltpu.sync_copy(idx_hbm.at[pl.ds(wid * CH, CH)], idx_v)
        pltpu.sync_copy(pts_hbm.at[pl.ds(wid * BPW, BPW)], pts_v)

        def start_gather(q, j):
            pltpu.async_copy(table_hbm.at[idx_v.at[q, 0]], rows[j], gsem[j])

        def wait_gather(q, j):
            pltpu.make_async_copy(
                table_hbm.at[idx_v.at[q, 0]], rows[j], gsem[j]
            ).wait()

        def out_slice(q):
            return out_hbm.at[:, pl.ds(wid * BPW + q * WIN_B, WIN_B), :]

        start_gather(0, 0)
        start_gather(1, 1)

        @pl.loop(0, CH, step=2)
        def _(qq):
            for j in range(2):
                q = qq + j
                wait_gather(q, j)

                @pl.when(q >= 2)
                def _():
                    pltpu.make_async_copy(
                        obuf[j], out_slice(q - 2), osem[j]
                    ).wait()

                for b in range(WIN_B):
                    bb = q * WIN_B + b
                    pvecs = [pts_v[bb, pl.ds(d * L, L)] for d in range(ND)]

                    @pl.loop(0, K)
                    def _(k, b=b, pvecs=pvecs):
                        r = b * K + k
                        for d in range(ND):
                            sl = pl.ds(d * L, L)
                            obuf[j][k, b, sl] = rows[j][r, sl] - pvecs[d]

                pltpu.async_copy(obuf[j], out_slice(q), osem[j])

                @pl.when(q + 2 < CH)
                def _():
                    start_gather(q + 2, j)

        for j in range(2):
            pltpu.make_async_copy(
                obuf[j], out_slice(CH - 2 + j), osem[j]
            ).wait()

    return run


_sc_kernel = _make_sc_kernel()


def kernel(indices, points, sampleLocs):
    idx_flat = indices.astype(jnp.int32).reshape(B // WIN_B, 1, WIN)
    out_kmajor = _sc_kernel(sampleLocs, idx_flat, points)
    return jnp.transpose(out_kmajor, (1, 0, 2))


# speedup vs baseline: 9.9498x; 2.7794x over previous
import functools

import jax
import jax.numpy as jnp
from jax import lax
from jax.experimental import pallas as pl
from jax.experimental.pallas import tpu as pltpu
from jax.experimental.pallas import tpu_sc as plsc

B = 4096
K = 50
M = 100000
D = 128
L = 16
ND = D // L
NW = 32
WIN_B = 4
WIN = WIN_B * K
NSUB = 2
SUBW = WIN // NSUB
BPW = B // NW
CH = BPW // WIN_B
NBUF = 4


def _make_sc_kernel():
    mesh = plsc.VectorSubcoreMesh(
        core_axis_name="core", subcore_axis_name="subcore"
    )

    @functools.partial(
        pl.kernel,
        out_type=jax.ShapeDtypeStruct((K, B, D), jnp.float32),
        mesh=mesh,
        scratch_types=[
            pltpu.VMEM((CH, NSUB, SUBW), jnp.int32),
            pltpu.VMEM((BPW, D), jnp.float32),
            pltpu.VMEM((WIN, D), jnp.float32),
            pltpu.VMEM((WIN, D), jnp.float32),
            pltpu.VMEM((WIN, D), jnp.float32),
            pltpu.VMEM((WIN, D), jnp.float32),
            pltpu.SemaphoreType.DMA,
            pltpu.SemaphoreType.DMA,
            pltpu.SemaphoreType.DMA,
            pltpu.SemaphoreType.DMA,
            pltpu.SemaphoreType.DMA,
            pltpu.SemaphoreType.DMA,
            pltpu.SemaphoreType.DMA,
            pltpu.SemaphoreType.DMA,
        ],
        compiler_params=pltpu.CompilerParams(use_tc_tiling_on_sc=False),
    )
    def run(table_hbm, idx_hbm, pts_hbm, out_hbm,
            idx_v, pts_v, r0, r1, r2, r3,
            gs0, gs1, gs2, gs3, os0, os1, os2, os3):
        rows = (r0, r1, r2, r3)
        gsem = (gs0, gs1, gs2, gs3)
        osem = (os0, os1, os2, os3)

        wid = lax.axis_index("core") * 16 + lax.axis_index("subcore")

        pltpu.sync_copy(idx_hbm.at[pl.ds(wid * CH, CH)], idx_v)
        pltpu.sync_copy(pts_hbm.at[pl.ds(wid * BPW, BPW)], pts_v)

        def start_gather(q, j):
            for s in range(NSUB):
                pltpu.async_copy(
                    table_hbm.at[idx_v.at[q, s]],
                    rows[j].at[pl.ds(s * SUBW, SUBW)],
                    gsem[j],
                )

        def wait_gather(q, j):
            for s in range(NSUB):
                pltpu.make_async_copy(
                    table_hbm.at[idx_v.at[q, s]],
                    rows[j].at[pl.ds(s * SUBW, SUBW)],
                    gsem[j],
                ).wait()

        def wb_pair(q, j, b):
            src = rows[j].at[pl.ds(b * K, K)]
            dst = out_hbm.at[:, wid * BPW + q * WIN_B + b, :]
            return src, dst

        start_gather(0, 0)
        start_gather(1, 1)

        @pl.loop(0, CH, step=NBUF)
        def _(qq):
            for j in range(NBUF):
                q = qq + j
                wait_gather(q, j)

                j2 = (j + 2) % NBUF

                @pl.when(q + 2 < CH)
                def _():
                    @pl.when(q >= 2)
                    def _():
                        for b in range(WIN_B):
                            src, dst = wb_pair(q - 2, j2, b)
                            pltpu.make_async_copy(src, dst, osem[j2]).wait()

                    start_gather(q + 2, j2)

                for b in range(WIN_B):
                    bb = q * WIN_B + b
                    pvecs = [pts_v[bb, pl.ds(d * L, L)] for d in range(ND)]

                    @plsc.parallel_loop(0, K, unroll=2)
                    def _(k, b=b, pvecs=pvecs):
                        r = b * K + k
                        for d in range(ND):
                            sl = pl.ds(d * L, L)
                            rows[j][r, sl] = rows[j][r, sl] - pvecs[d]

                for b in range(WIN_B):
                    src, dst = wb_pair(q, j, b)
                    pltpu.async_copy(src, dst, osem[j])

        for jj in range(NBUF):
            q = CH - NBUF + jj
            for b in range(WIN_B):
                src, dst = wb_pair(q, jj, b)
                pltpu.make_async_copy(src, dst, osem[jj]).wait()

    return run


_sc_kernel = _make_sc_kernel()


def kernel(indices, points, sampleLocs):
    idx_flat = indices.astype(jnp.int32).reshape(B // WIN_B, NSUB, SUBW)
    out_kmajor = _sc_kernel(sampleLocs, idx_flat, points)
    return jnp.transpose(out_kmajor, (1, 0, 2))
